# Initial kernel scaffold; baseline (speedup 1.0000x reference)
import jax
import jax.numpy as jnp
from jax.experimental import pallas as pl


def kernel(x, mlp_user_w, mlp_item_w, gmf_user_w, gmf_item_w, W1, b1, W2, b2, W3, b3, W_last, b_last):
    raise NotImplementedError("write your pallas kernel here")


# speedup vs baseline: 1.4526x; 1.4526x over previous
import functools

import jax
import jax.numpy as jnp
from jax import lax
from jax.experimental import pallas as pl
from jax.experimental.pallas import tpu as pltpu
from jax.experimental.pallas import tpu_sc as plsc

_B = 16384
_D = 128
_NW = 32
_BPW = _B // _NW
_CH = 128
_NCH = _BPW // _CH


def _sc_gather(uidx2, iidx2, mu, mi, gu, gi):
    mesh = plsc.VectorSubcoreMesh(core_axis_name="c", subcore_axis_name="s")
    f32 = jnp.float32

    @functools.partial(
        pl.kernel,
        out_type=(
            jax.ShapeDtypeStruct((_B, _D), f32),
            jax.ShapeDtypeStruct((_B, _D), f32),
            jax.ShapeDtypeStruct((_B, _D), f32),
            jax.ShapeDtypeStruct((_B, _D), f32),
        ),
        mesh=mesh,
        scratch_types=(
            pltpu.VMEM((_NCH, _CH), jnp.int32),
            pltpu.VMEM((_NCH, _CH), jnp.int32),
            pltpu.VMEM((_CH, _D), f32),
            pltpu.SemaphoreType.DMA,
        ),
    )
    def run(uidx_h, iidx_h, mu_h, mi_h, gu_h, gi_h,
            out_mu, out_mi, out_gu, out_gi,
            uidx_v, iidx_v, rows, sem):
        wid = lax.axis_index("s") * 2 + lax.axis_index("c")
        pltpu.sync_copy(uidx_h.at[pl.ds(wid * _NCH, _NCH)], uidx_v)
        pltpu.sync_copy(iidx_h.at[pl.ds(wid * _NCH, _NCH)], iidx_v)
        for j in range(_NCH):
            ob = wid * _BPW + j * _CH
            for tab, idxv, out in ((mu_h, uidx_v, out_mu),
                                   (mi_h, iidx_v, out_mi),
                                   (gu_h, uidx_v, out_gu),
                                   (gi_h, iidx_v, out_gi)):
                pltpu.async_copy(tab.at[idxv.at[j]], rows, sem).wait()
                pltpu.sync_copy(rows, out.at[pl.ds(ob, _CH)])

    return run(uidx2, iidx2, mu, mi, gu, gi)


_BLK = 512


def _tc_dense(um, im, ug, ig, w1a, w1b, b1, w2t, b2, w3t, b3, wg, wm, blast):
    f32 = jnp.float32

    def body(um_r, im_r, ug_r, ig_r, w1a_r, w1b_r, b1_r, w2_r, b2_r,
             w3_r, b3_r, wg_r, wm_r, bl_r, out_r):
        dot = functools.partial(lax.dot_general,
                                dimension_numbers=(((1,), (0,)), ((), ())),
                                preferred_element_type=f32,
                                precision=lax.Precision.HIGHEST)
        h = jnp.maximum(dot(um_r[...], w1a_r[...]) + dot(im_r[...], w1b_r[...])
                        + b1_r[...], 0.0)
        h = jnp.maximum(dot(h, w2_r[...]) + b2_r[...], 0.0)
        h = jnp.maximum(dot(h, w3_r[...]) + b3_r[...], 0.0)
        g = jnp.sum(ug_r[...] * ig_r[...] * wg_r[...], axis=1, keepdims=True)
        out_r[...] = g + jnp.sum(h * wm_r[...], axis=1, keepdims=True) + bl_r[...]

    full = lambda shape: pl.BlockSpec(shape, lambda i: (0,) * len(shape))
    batch = pl.BlockSpec((_BLK, _D), lambda i: (i, 0))
    return pl.pallas_call(
        body,
        grid=(_B // _BLK,),
        in_specs=[batch, batch, batch, batch,
                  full((_D, 64)), full((_D, 64)), full((1, 64)),
                  full((64, 16)), full((1, 16)),
                  full((16, 8)), full((1, 8)),
                  full((1, _D)), full((1, 8)), full((1, 1))],
        out_specs=pl.BlockSpec((_BLK, 1), lambda i: (i, 0)),
        out_shape=jax.ShapeDtypeStruct((_B, 1), f32),
    )(um, im, ug, ig, w1a, w1b, b1, w2t, b2, w3t, b3, wg, wm, blast)


def kernel(x, mlp_user_w, mlp_item_w, gmf_user_w, gmf_item_w,
           W1, b1, W2, b2, W3, b3, W_last, b_last):
    uidx2 = x[:, 0].reshape(_B // _CH, _CH)
    iidx2 = x[:, 1].reshape(_B // _CH, _CH)
    um, im, ug, ig = _sc_gather(uidx2, iidx2,
                                mlp_user_w, mlp_item_w, gmf_user_w, gmf_item_w)
    out = _tc_dense(
        um, im, ug, ig,
        W1[:, :_D].T, W1[:, _D:].T, b1.reshape(1, 64),
        W2.T, b2.reshape(1, 16),
        W3.T, b3.reshape(1, 8),
        W_last[:, :_D], W_last[:, _D:], b_last.reshape(1, 1))
    return out
